# baseline (speedup 1.0000x reference)
import functools

import jax
import jax.numpy as jnp
from jax import lax
from jax.experimental import pallas as pl
from jax.experimental.pallas import tpu as pltpu
from jax.experimental.pallas import tpu_sc as plsc

NU = 50000
E = 128000
R = 5
D_IN = 128
MSG_R = 32
OUT = 64

NC = 2
NS = 16

CH = 128
NCHT = 63
EP = NS * NCHT * CH
PAD = EP - E
BLK = 3
NBLK = NCHT // BLK
ANU = NU + CH

ZCH = 80
NZC = NU // ZCH
ZPT = -(-NZC // NS)

BN = 2000
NB = NU // BN


def _proj_body(feat_ref, w_ref, cj_ref, out_ref):
    x = jnp.dot(feat_ref[...], w_ref[0], preferred_element_type=jnp.float32)
    out_ref[...] = x * cj_ref[...]


def _project(feat, w, cj):
    return pl.pallas_call(
        _proj_body,
        grid=(NB, R),
        in_specs=[
            pl.BlockSpec((BN, D_IN), lambda nb, r: (nb, 0)),
            pl.BlockSpec((1, D_IN, MSG_R), lambda nb, r: (r, 0, 0)),
            pl.BlockSpec((BN, 1), lambda nb, r: (nb, 0)),
        ],
        out_specs=pl.BlockSpec((BN, MSG_R), lambda nb, r: (r * NB + nb, 0)),
        out_shape=jax.ShapeDtypeStruct((R * NU, MSG_R), jnp.float32),
    )(feat, w, cj)


def _fc_body(h_ref, w_ref, ci_ref, b_ref, out_ref):
    acc = jnp.dot(h_ref[0], w_ref[0], preferred_element_type=jnp.float32)
    for r in range(1, R):
        acc += jnp.dot(h_ref[r], w_ref[r], preferred_element_type=jnp.float32)
    out_ref[...] = acc * ci_ref[...] + b_ref[...]


def _fc(h3, wfc, ci, b):
    return pl.pallas_call(
        _fc_body,
        grid=(NB,),
        in_specs=[
            pl.BlockSpec((R, BN, MSG_R), lambda nb: (0, nb, 0)),
            pl.BlockSpec((R, MSG_R, OUT), lambda nb: (0, 0, 0)),
            pl.BlockSpec((BN, 1), lambda nb: (nb, 0)),
            pl.BlockSpec((1, OUT), lambda nb: (0, 0)),
        ],
        out_specs=pl.BlockSpec((BN, OUT), lambda nb: (nb, 0)),
        out_shape=jax.ShapeDtypeStruct((NU, OUT), jnp.float32),
    )(h3, wfc, ci, b)


def _sc_segment_sums(xu, xi, gidx3, sidx3):
    mesh = plsc.VectorSubcoreMesh(
        core_axis_name="c", subcore_axis_name="s", num_cores=NC, num_subcores=NS
    )

    @functools.partial(
        pl.kernel,
        out_type=(
            jax.ShapeDtypeStruct((R * NU, MSG_R), jnp.float32),
            jax.ShapeDtypeStruct((R * NU, MSG_R), jnp.float32),
        ),
        mesh=mesh,
        scratch_types=[
            pltpu.VMEM_SHARED((ANU, MSG_R), jnp.float32),
            pltpu.VMEM((2, BLK, CH), jnp.int32),
            pltpu.VMEM((2, BLK, CH), jnp.int32),
            pltpu.VMEM((BLK, CH, MSG_R), jnp.float32),
            pltpu.VMEM((BLK, CH, MSG_R), jnp.float32),
            pltpu.VMEM((ZCH, MSG_R), jnp.float32),
            pltpu.SemaphoreType.DMA,
            pltpu.SemaphoreType.DMA,
            pltpu.SemaphoreType.DMA,
            pltpu.SemaphoreType.DMA,
        ],
        compiler_params=pltpu.CompilerParams(use_tc_tiling_on_sc=False),
    )
    def kern(xu_h, xi_h, gidx_h, sidx_h, hi_h, hu_h,
             acc, gidx_v, sidx_v, rows0, rows1, zeros_v,
             isem, gsem, ssem, osem):
        core = lax.axis_index("c")
        sid = lax.axis_index("s")

        @pl.loop(0, ZCH)
        def _zinit(zi):
            zeros_v[zi, pl.ds(0, 16)] = jnp.zeros((16,), jnp.float32)
            zeros_v[zi, pl.ds(16, 16)] = jnp.zeros((16,), jnp.float32)

        def fire_zero():
            @pl.loop(0, ZPT)
            def _z(z):
                c = z * NS + sid

                @pl.when(c < NZC)
                def _():
                    pltpu.async_copy(zeros_v, acc.at[pl.ds(c * ZCH, ZCH)], osem)

        def drain_zero():
            @pl.loop(0, ZPT)
            def _z(z):
                c = z * NS + sid

                @pl.when(c < NZC)
                def _():
                    pltpu.make_async_copy(
                        zeros_v, acc.at[pl.ds(0, ZCH)], osem).wait()

        fire_zero()
        drain_zero()
        plsc.subcore_barrier()

        def run(table, out, base_t):
            @pl.loop(0, R)
            def _task(i):
                t = (base_t + i) * NS + sid

                def fire_idx(b, p):
                    pltpu.async_copy(
                        gidx_h.at[t, pl.ds(b * BLK, BLK)], gidx_v.at[p], isem)
                    pltpu.async_copy(
                        sidx_h.at[t, pl.ds(b * BLK, BLK)], sidx_v.at[p], isem)

                def drain_idx():
                    pltpu.make_async_copy(
                        gidx_h.at[0, pl.ds(0, BLK)], gidx_v.at[0], isem).wait()
                    pltpu.make_async_copy(
                        sidx_h.at[0, pl.ds(0, BLK)], sidx_v.at[0], isem).wait()

                def fire_gathers(p, rbuf):
                    for k in range(BLK):
                        pltpu.async_copy(
                            table.at[gidx_v.at[p, k]], rbuf.at[k], gsem)

                def drain_gathers(rbuf):
                    for k in range(BLK):
                        pltpu.make_async_copy(
                            table.at[gidx_v.at[0, 0]], rbuf.at[k], gsem).wait()

                def fire_scatters(p, rbuf):
                    for k in range(BLK):
                        pltpu.async_copy(
                            rbuf.at[k], acc.at[sidx_v.at[p, k]], ssem,
                            add=True)

                def drain_scatters(rbuf):
                    for k in range(BLK):
                        pltpu.make_async_copy(
                            rbuf.at[k], acc.at[sidx_v.at[0, 0]], ssem).wait()

                fire_idx(0, 0)
                drain_idx()
                fire_idx(1, 1)
                fire_gathers(0, rows0)

                @pl.loop(0, (NBLK - 1) // 2)
                def _blk(s):
                    drain_gathers(rows0)
                    drain_idx()
                    fire_gathers(1, rows1)
                    fire_scatters(0, rows0)
                    drain_scatters(rows0)
                    fire_idx(2 * s + 2, 0)
                    drain_gathers(rows1)
                    drain_idx()
                    fire_gathers(0, rows0)
                    fire_scatters(1, rows1)
                    drain_scatters(rows1)

                    @pl.when(s < (NBLK - 1) // 2 - 1)
                    def _():
                        fire_idx(2 * s + 3, 1)

                drain_gathers(rows0)
                fire_scatters(0, rows0)
                drain_scatters(rows0)

                plsc.subcore_barrier()

                @pl.loop(0, ZPT)
                def _o1(z):
                    c = z * NS + sid

                    @pl.when(c < NZC)
                    def _():
                        pltpu.async_copy(
                            acc.at[pl.ds(c * ZCH, ZCH)],
                            out.at[pl.ds(i * NU + c * ZCH, ZCH)], osem)

                @pl.loop(0, ZPT)
                def _o2(z):
                    c = z * NS + sid

                    @pl.when(c < NZC)
                    def _():
                        pltpu.make_async_copy(
                            acc.at[pl.ds(0, ZCH)],
                            out.at[pl.ds(0, ZCH)], osem).wait()

                fire_zero()
                drain_zero()
                plsc.subcore_barrier()

        @pl.when(core == 0)
        def _c0():
            run(xu_h, hi_h, 0)

        @pl.when(core == 1)
        def _c1():
            run(xi_h, hu_h, R)

    return kern(xu, xi, gidx3, sidx3)


def kernel(ufeat, ifeat, cj_user, cj_movie, ci_user, ci_movie, W_r, W_rev,
           ufc_W, ufc_b, ifc_W, ifc_b,
           edge_index_0, edge_index_1, edge_index_2, edge_index_3, edge_index_4):
    edges = [edge_index_0, edge_index_1, edge_index_2, edge_index_3, edge_index_4]
    src = jnp.stack([e[0] for e in edges])
    dst = jnp.stack([e[1] for e in edges])
    offs = (jnp.arange(R, dtype=jnp.int32) * NU)[:, None]
    gidx = jnp.concatenate([src + offs, dst + offs], axis=0)
    sidx = jnp.concatenate([dst, src], axis=0)
    padg = jnp.zeros((2 * R, PAD), jnp.int32)
    pads = jnp.broadcast_to(
        NU + (jnp.arange(PAD, dtype=jnp.int32) % CH), (2 * R, PAD))
    gidx3 = jnp.concatenate([gidx, padg], axis=1).reshape(2 * R * NS, NCHT, CH)
    sidx3 = jnp.concatenate([sidx, pads], axis=1).reshape(2 * R * NS, NCHT, CH)

    xu = (jnp.einsum('nd,rdk->rnk', ufeat, W_r) * cj_user[None]).reshape(R * NU, MSG_R)
    xi = (jnp.einsum('nd,rdk->rnk', ifeat, W_rev) * cj_movie[None]).reshape(R * NU, MSG_R)

    hi, hu = _sc_segment_sums(xu, xi, gidx3, sidx3)

    u_out = ci_user * jnp.einsum('rnk,rko->no', hu.reshape(R, NU, MSG_R), ufc_W.reshape(R, MSG_R, OUT)) + ufc_b
    i_out = ci_movie * jnp.einsum('rnk,rko->no', hi.reshape(R, NU, MSG_R), ifc_W.reshape(R, MSG_R, OUT)) + ifc_b
    return (u_out, i_out)


# speedup vs baseline: 1.0238x; 1.0238x over previous
import functools

import jax
import jax.numpy as jnp
from jax import lax
from jax.experimental import pallas as pl
from jax.experimental.pallas import tpu as pltpu
from jax.experimental.pallas import tpu_sc as plsc

NU = 50000
E = 128000
R = 5
D_IN = 128
MSG_R = 32
OUT = 64

NC = 2
NS = 16

CH = 128
NCHT = 63
EP = NS * NCHT * CH
PAD = EP - E
BLK = 3
NBLK = NCHT // BLK

PK = 4
NQ = 12504
TNU = PK * NQ
BN = 4168
NB = NQ // BN

ANU = TNU + CH

ZCH = 96
NZC = TNU // ZCH
ZPT = -(-NZC // NS)


def _proj_body(f0, f1, f2, f3, cjp_ref, w_ref, out_ref):
    rowid = (jax.lax.broadcasted_iota(jnp.int32, (BN, D_IN), 0)
             + pl.program_id(0) * BN)
    f3m = jnp.where(rowid < NU - (PK - 1) * NQ, f3[...], 0.0)
    fs = (f0[...], f1[...], f2[...], f3m)
    x = jnp.dot(fs[0], w_ref[0, 0:D_IN],
                preferred_element_type=jnp.float32)
    for a in range(1, PK):
        x += jnp.dot(fs[a], w_ref[0, a * D_IN:(a + 1) * D_IN],
                     preferred_element_type=jnp.float32)
    out_ref[...] = x * cjp_ref[...]


def _project(feat, wblk, cjp):
    feat_specs = [
        pl.BlockSpec((BN, D_IN), lambda nb, r, a=a: (a * NB + nb, 0))
        for a in range(PK)
    ]
    return pl.pallas_call(
        _proj_body,
        grid=(NB, R),
        in_specs=feat_specs + [
            pl.BlockSpec((BN, PK * MSG_R), lambda nb, r: (nb, 0)),
            pl.BlockSpec((1, PK * D_IN, PK * MSG_R), lambda nb, r: (r, 0, 0)),
        ],
        out_specs=pl.BlockSpec((BN, PK * MSG_R), lambda nb, r: (r * NB + nb, 0)),
        out_shape=jax.ShapeDtypeStruct((R * NQ, PK * MSG_R), jnp.float32),
    )(feat, feat, feat, feat, cjp, wblk)


def _fc_body(h_ref, w_ref, c0, c1, c2, c3, b_ref, o0, o1, o2, o3):
    r = pl.program_id(1)
    cs = (c0, c1, c2, c3)
    outs = (o0, o1, o2, o3)
    for a in range(PK):
        part = jnp.dot(h_ref[...], w_ref[0, :, a * OUT:(a + 1) * OUT],
                       preferred_element_type=jnp.float32)

        @pl.when(r == 0)
        def _():
            outs[a][...] = part

        @pl.when(r > 0)
        def _():
            outs[a][...] += part

    @pl.when(r == R - 1)
    def _():
        for a in range(PK):
            outs[a][...] = outs[a][...] * cs[a][...] + b_ref[...]


def _fc(hp, wblk, ci, b):
    ci_specs = [
        pl.BlockSpec((BN, 1), lambda nb, r, a=a: (a * NB + nb, 0))
        for a in range(PK)
    ]
    outs = pl.pallas_call(
        _fc_body,
        grid=(NB, R),
        in_specs=[
            pl.BlockSpec((BN, PK * MSG_R), lambda nb, r: (r * NB + nb, 0)),
            pl.BlockSpec((1, PK * MSG_R, PK * OUT), lambda nb, r: (r, 0, 0)),
        ] + ci_specs + [
            pl.BlockSpec((1, OUT), lambda nb, r: (0, 0)),
        ],
        out_specs=[pl.BlockSpec((BN, OUT), lambda nb, r: (nb, 0))] * PK,
        out_shape=[jax.ShapeDtypeStruct((NQ, OUT), jnp.float32)] * PK,
    )(hp, wblk, ci, ci, ci, ci, b)
    return jnp.concatenate(
        list(outs[:PK - 1]) + [outs[PK - 1][:NU - (PK - 1) * NQ]], axis=0)


def _sc_segment_sums(xu, xi, gidx3, sidx3):
    mesh = plsc.VectorSubcoreMesh(
        core_axis_name="c", subcore_axis_name="s", num_cores=NC, num_subcores=NS
    )

    @functools.partial(
        pl.kernel,
        out_type=(
            jax.ShapeDtypeStruct((R * TNU, MSG_R), jnp.float32),
            jax.ShapeDtypeStruct((R * TNU, MSG_R), jnp.float32),
        ),
        mesh=mesh,
        scratch_types=[
            pltpu.VMEM_SHARED((ANU, MSG_R), jnp.float32),
            pltpu.VMEM((2, BLK, CH), jnp.int32),
            pltpu.VMEM((2, BLK, CH), jnp.int32),
            pltpu.VMEM((BLK, CH, MSG_R), jnp.float32),
            pltpu.VMEM((BLK, CH, MSG_R), jnp.float32),
            pltpu.VMEM((ZCH, MSG_R), jnp.float32),
            pltpu.SemaphoreType.DMA,
            pltpu.SemaphoreType.DMA,
            pltpu.SemaphoreType.DMA,
            pltpu.SemaphoreType.DMA,
        ],
        compiler_params=pltpu.CompilerParams(use_tc_tiling_on_sc=False),
    )
    def kern(xu_h, xi_h, gidx_h, sidx_h, hi_h, hu_h,
             acc, gidx_v, sidx_v, rows0, rows1, zeros_v,
             isem, gsem, ssem, osem):
        core = lax.axis_index("c")
        sid = lax.axis_index("s")

        @pl.loop(0, ZCH)
        def _zinit(zi):
            zeros_v[zi, pl.ds(0, 16)] = jnp.zeros((16,), jnp.float32)
            zeros_v[zi, pl.ds(16, 16)] = jnp.zeros((16,), jnp.float32)

        def fire_zero():
            @pl.loop(0, ZPT)
            def _z(z):
                c = z * NS + sid

                @pl.when(c < NZC)
                def _():
                    pltpu.async_copy(zeros_v, acc.at[pl.ds(c * ZCH, ZCH)], osem)

        def drain_zero():
            @pl.loop(0, ZPT)
            def _z(z):
                c = z * NS + sid

                @pl.when(c < NZC)
                def _():
                    pltpu.make_async_copy(
                        zeros_v, acc.at[pl.ds(0, ZCH)], osem).wait()

        fire_zero()
        drain_zero()
        plsc.subcore_barrier()

        def run(table, out, base_t):
            @pl.loop(0, R)
            def _task(i):
                t = (base_t + i) * NS + sid

                def fire_idx(b, p):
                    pltpu.async_copy(
                        gidx_h.at[t, pl.ds(b * BLK, BLK)], gidx_v.at[p], isem)
                    pltpu.async_copy(
                        sidx_h.at[t, pl.ds(b * BLK, BLK)], sidx_v.at[p], isem)

                def drain_idx():
                    pltpu.make_async_copy(
                        gidx_h.at[0, pl.ds(0, BLK)], gidx_v.at[0], isem).wait()
                    pltpu.make_async_copy(
                        sidx_h.at[0, pl.ds(0, BLK)], sidx_v.at[0], isem).wait()

                def fire_gathers(p, rbuf):
                    for k in range(BLK):
                        pltpu.async_copy(
                            table.at[gidx_v.at[p, k]], rbuf.at[k], gsem)

                def drain_gathers(rbuf):
                    for k in range(BLK):
                        pltpu.make_async_copy(
                            table.at[gidx_v.at[0, 0]], rbuf.at[k], gsem).wait()

                def fire_scatters(p, rbuf):
                    for k in range(BLK):
                        pltpu.async_copy(
                            rbuf.at[k], acc.at[sidx_v.at[p, k]], ssem,
                            add=True)

                def drain_scatters(rbuf):
                    for k in range(BLK):
                        pltpu.make_async_copy(
                            rbuf.at[k], acc.at[sidx_v.at[0, 0]], ssem).wait()

                fire_idx(0, 0)
                drain_idx()
                fire_idx(1, 1)
                fire_gathers(0, rows0)

                @pl.loop(0, (NBLK - 1) // 2)
                def _blk(s):
                    drain_gathers(rows0)
                    drain_idx()
                    fire_gathers(1, rows1)
                    fire_scatters(0, rows0)
                    drain_scatters(rows0)
                    fire_idx(2 * s + 2, 0)
                    drain_gathers(rows1)
                    drain_idx()
                    fire_gathers(0, rows0)
                    fire_scatters(1, rows1)
                    drain_scatters(rows1)

                    @pl.when(s < (NBLK - 1) // 2 - 1)
                    def _():
                        fire_idx(2 * s + 3, 1)

                drain_gathers(rows0)
                fire_scatters(0, rows0)
                drain_scatters(rows0)

                plsc.subcore_barrier()

                @pl.loop(0, ZPT)
                def _o1(z):
                    c = z * NS + sid

                    @pl.when(c < NZC)
                    def _():
                        pltpu.async_copy(
                            acc.at[pl.ds(c * ZCH, ZCH)],
                            out.at[pl.ds(i * TNU + c * ZCH, ZCH)], osem)

                @pl.loop(0, ZPT)
                def _o2(z):
                    c = z * NS + sid

                    @pl.when(c < NZC)
                    def _():
                        pltpu.make_async_copy(
                            acc.at[pl.ds(0, ZCH)],
                            out.at[pl.ds(0, ZCH)], osem).wait()

                fire_zero()
                drain_zero()
                plsc.subcore_barrier()

        @pl.when(core == 0)
        def _c0():
            run(xu_h, hi_h, 0)

        @pl.when(core == 1)
        def _c1():
            run(xi_h, hu_h, R)

    return kern(xu, xi, gidx3, sidx3)


def kernel(ufeat, ifeat, cj_user, cj_movie, ci_user, ci_movie, W_r, W_rev,
           ufc_W, ufc_b, ifc_W, ifc_b,
           edge_index_0, edge_index_1, edge_index_2, edge_index_3, edge_index_4):
    edges = [edge_index_0, edge_index_1, edge_index_2, edge_index_3, edge_index_4]
    src = jnp.stack([e[0] for e in edges])
    dst = jnp.stack([e[1] for e in edges])

    def perm(n):
        return PK * (n % NQ) + n // NQ

    psrc = perm(src)
    pdst = perm(dst)
    offs = (jnp.arange(R, dtype=jnp.int32) * TNU)[:, None]
    gidx = jnp.concatenate([psrc + offs, pdst + offs], axis=0)
    sidx = jnp.concatenate([pdst, psrc], axis=0)
    padg = jnp.zeros((2 * R, PAD), jnp.int32)
    pads = jnp.broadcast_to(
        TNU + (jnp.arange(PAD, dtype=jnp.int32) % CH), (2 * R, PAD))
    gidx3 = jnp.concatenate([gidx, padg], axis=1).reshape(2 * R * NS, NCHT, CH)
    sidx3 = jnp.concatenate([sidx, pads], axis=1).reshape(2 * R * NS, NCHT, CH)

    eye = jnp.eye(PK, dtype=jnp.float32)[None, :, None, :, None]
    wblk_u = (eye * W_r[:, None, :, None, :]).reshape(R, PK * D_IN, PK * MSG_R)
    wblk_i = (eye * W_rev[:, None, :, None, :]).reshape(R, PK * D_IN, PK * MSG_R)
    wfc_u = (eye * ufc_W.reshape(R, MSG_R, OUT)[:, None, :, None, :]
             ).reshape(R, PK * MSG_R, PK * OUT)
    wfc_i = (eye * ifc_W.reshape(R, MSG_R, OUT)[:, None, :, None, :]
             ).reshape(R, PK * MSG_R, PK * OUT)
    zpad = jnp.zeros((TNU - NU, 1), jnp.float32)
    cjp_u = jnp.repeat(
        jnp.concatenate([cj_user, zpad]).reshape(PK, NQ).T, MSG_R, axis=1)
    cjp_i = jnp.repeat(
        jnp.concatenate([cj_movie, zpad]).reshape(PK, NQ).T, MSG_R, axis=1)

    xu = _project(ufeat, wblk_u, cjp_u)
    xi = _project(ifeat, wblk_i, cjp_i)

    hi, hu = _sc_segment_sums(
        xu.reshape(R * TNU, MSG_R), xi.reshape(R * TNU, MSG_R), gidx3, sidx3)

    u_out = _fc(hu.reshape(R * NQ, PK * MSG_R), wfc_u, ci_user,
                ufc_b.reshape(1, OUT))
    i_out = _fc(hi.reshape(R * NQ, PK * MSG_R), wfc_i, ci_movie,
                ifc_b.reshape(1, OUT))
    return (u_out, i_out)


# speedup vs baseline: 1.1572x; 1.1303x over previous
import functools

import jax
import jax.numpy as jnp
from jax import lax
from jax.experimental import pallas as pl
from jax.experimental.pallas import tpu as pltpu
from jax.experimental.pallas import tpu_sc as plsc

NU = 50000
E = 128000
R = 5
D_IN = 128
MSG_R = 32
OUT = 64

NC = 2
NS = 16

CH = 128
NCHT = 63
EP = NS * NCHT * CH
PAD = EP - E
BLK = 3
NBLK = NCHT // BLK

PK = 4
NQ = 12504
TNU = PK * NQ
BN = 4168
NB = NQ // BN

ANU = TNU + CH

ZCH = 96
NZC = TNU // ZCH
ZPT = -(-NZC // NS)


def _proj_body(f0, f1, f2, f3, cjp_ref, w_ref, out_ref):
    rowid = (jax.lax.broadcasted_iota(jnp.int32, (BN, D_IN), 0)
             + pl.program_id(0) * BN)
    f3m = jnp.where(rowid < NU - (PK - 1) * NQ, f3[...], 0.0)
    fs = (f0[...], f1[...], f2[...], f3m)
    x = jnp.dot(fs[0], w_ref[0, 0:D_IN],
                preferred_element_type=jnp.float32)
    for a in range(1, PK):
        x += jnp.dot(fs[a], w_ref[0, a * D_IN:(a + 1) * D_IN],
                     preferred_element_type=jnp.float32)
    out_ref[...] = x * cjp_ref[...]


def _project(feat, wblk, cjp):
    feat_specs = [
        pl.BlockSpec((BN, D_IN), lambda nb, r, a=a: (a * NB + nb, 0))
        for a in range(PK)
    ]
    return pl.pallas_call(
        _proj_body,
        grid=(NB, R),
        in_specs=feat_specs + [
            pl.BlockSpec((BN, PK * MSG_R), lambda nb, r: (nb, 0)),
            pl.BlockSpec((1, PK * D_IN, PK * MSG_R), lambda nb, r: (r, 0, 0)),
        ],
        out_specs=pl.BlockSpec((BN, PK * MSG_R), lambda nb, r: (r * NB + nb, 0)),
        out_shape=jax.ShapeDtypeStruct((R * NQ, PK * MSG_R), jnp.float32),
    )(feat, feat, feat, feat, cjp, wblk)


def _fc_body(h_ref, w_ref, ci_ref, b_ref, o_ref):
    r = pl.program_id(2)
    part = jnp.dot(h_ref[...], w_ref[0, 0],
                   preferred_element_type=jnp.float32)

    @pl.when(r == 0)
    def _():
        o_ref[...] = part

    @pl.when(jnp.logical_and(r > 0, r < R - 1))
    def _():
        o_ref[...] += part

    @pl.when(r == R - 1)
    def _():
        o_ref[...] = (o_ref[...] + part) * ci_ref[...] + b_ref[...]


def _fc(hp, wblk4, ci, b):
    return pl.pallas_call(
        _fc_body,
        grid=(NB, PK, R),
        in_specs=[
            pl.BlockSpec((BN, PK * MSG_R), lambda nb, a, r: (r * NB + nb, 0)),
            pl.BlockSpec((1, 1, PK * MSG_R, OUT), lambda nb, a, r: (r, a, 0, 0)),
            pl.BlockSpec((BN, 1), lambda nb, a, r: (a * NB + nb, 0)),
            pl.BlockSpec((1, OUT), lambda nb, a, r: (0, 0)),
        ],
        out_specs=pl.BlockSpec((BN, OUT), lambda nb, a, r: (a * NB + nb, 0)),
        out_shape=jax.ShapeDtypeStruct((NU, OUT), jnp.float32),
    )(hp, wblk4, ci, b)


def _sc_segment_sums(xu, xi, gidx3, sidx3):
    mesh = plsc.VectorSubcoreMesh(
        core_axis_name="c", subcore_axis_name="s", num_cores=NC, num_subcores=NS
    )

    @functools.partial(
        pl.kernel,
        out_type=(
            jax.ShapeDtypeStruct((R * TNU, MSG_R), jnp.float32),
            jax.ShapeDtypeStruct((R * TNU, MSG_R), jnp.float32),
        ),
        mesh=mesh,
        scratch_types=[
            pltpu.VMEM_SHARED((ANU, MSG_R), jnp.float32),
            pltpu.VMEM((2, BLK, CH), jnp.int32),
            pltpu.VMEM((2, BLK, CH), jnp.int32),
            pltpu.VMEM((BLK, CH, MSG_R), jnp.float32),
            pltpu.VMEM((BLK, CH, MSG_R), jnp.float32),
            pltpu.VMEM((ZCH, MSG_R), jnp.float32),
            pltpu.SemaphoreType.DMA,
            pltpu.SemaphoreType.DMA,
            pltpu.SemaphoreType.DMA,
            pltpu.SemaphoreType.DMA,
        ],
        compiler_params=pltpu.CompilerParams(use_tc_tiling_on_sc=False),
    )
    def kern(xu_h, xi_h, gidx_h, sidx_h, hi_h, hu_h,
             acc, gidx_v, sidx_v, rows0, rows1, zeros_v,
             isem, gsem, ssem, osem):
        core = lax.axis_index("c")
        sid = lax.axis_index("s")

        @pl.loop(0, ZCH)
        def _zinit(zi):
            zeros_v[zi, pl.ds(0, 16)] = jnp.zeros((16,), jnp.float32)
            zeros_v[zi, pl.ds(16, 16)] = jnp.zeros((16,), jnp.float32)

        def fire_zero():
            @pl.loop(0, ZPT)
            def _z(z):
                c = z * NS + sid

                @pl.when(c < NZC)
                def _():
                    pltpu.async_copy(zeros_v, acc.at[pl.ds(c * ZCH, ZCH)], osem)

        def drain_zero():
            @pl.loop(0, ZPT)
            def _z(z):
                c = z * NS + sid

                @pl.when(c < NZC)
                def _():
                    pltpu.make_async_copy(
                        zeros_v, acc.at[pl.ds(0, ZCH)], osem).wait()

        fire_zero()
        drain_zero()
        plsc.subcore_barrier()

        def run(table, out, base_t):
            @pl.loop(0, R)
            def _task(i):
                t = (base_t + i) * NS + sid

                def fire_idx(b, p):
                    pltpu.async_copy(
                        gidx_h.at[t, pl.ds(b * BLK, BLK)], gidx_v.at[p], isem)
                    pltpu.async_copy(
                        sidx_h.at[t, pl.ds(b * BLK, BLK)], sidx_v.at[p], isem)

                def drain_idx():
                    pltpu.make_async_copy(
                        gidx_h.at[0, pl.ds(0, BLK)], gidx_v.at[0], isem).wait()
                    pltpu.make_async_copy(
                        sidx_h.at[0, pl.ds(0, BLK)], sidx_v.at[0], isem).wait()

                def fire_gathers(p, rbuf):
                    for k in range(BLK):
                        pltpu.async_copy(
                            table.at[gidx_v.at[p, k]], rbuf.at[k], gsem)

                def drain_gathers(rbuf):
                    for k in range(BLK):
                        pltpu.make_async_copy(
                            table.at[gidx_v.at[0, 0]], rbuf.at[k], gsem).wait()

                def fire_scatters(p, rbuf):
                    for k in range(BLK):
                        pltpu.async_copy(
                            rbuf.at[k], acc.at[sidx_v.at[p, k]], ssem,
                            add=True)

                def drain_scatters(rbuf):
                    for k in range(BLK):
                        pltpu.make_async_copy(
                            rbuf.at[k], acc.at[sidx_v.at[0, 0]], ssem).wait()

                fire_idx(0, 0)
                drain_idx()
                fire_idx(1, 1)
                fire_gathers(0, rows0)

                @pl.loop(0, (NBLK - 1) // 2)
                def _blk(s):
                    drain_gathers(rows0)
                    drain_idx()
                    fire_gathers(1, rows1)
                    fire_scatters(0, rows0)
                    drain_scatters(rows0)
                    fire_idx(2 * s + 2, 0)
                    drain_gathers(rows1)
                    drain_idx()
                    fire_gathers(0, rows0)
                    fire_scatters(1, rows1)
                    drain_scatters(rows1)

                    @pl.when(s < (NBLK - 1) // 2 - 1)
                    def _():
                        fire_idx(2 * s + 3, 1)

                drain_gathers(rows0)
                fire_scatters(0, rows0)
                drain_scatters(rows0)

                plsc.subcore_barrier()

                @pl.loop(0, ZPT)
                def _o1(z):
                    c = z * NS + sid

                    @pl.when(c < NZC)
                    def _():
                        pltpu.async_copy(
                            acc.at[pl.ds(c * ZCH, ZCH)],
                            out.at[pl.ds(i * TNU + c * ZCH, ZCH)], osem)

                @pl.loop(0, ZPT)
                def _o2(z):
                    c = z * NS + sid

                    @pl.when(c < NZC)
                    def _():
                        pltpu.make_async_copy(
                            acc.at[pl.ds(0, ZCH)],
                            out.at[pl.ds(0, ZCH)], osem).wait()

                fire_zero()
                drain_zero()
                plsc.subcore_barrier()

        @pl.when(core == 0)
        def _c0():
            run(xu_h, hi_h, 0)

        @pl.when(core == 1)
        def _c1():
            run(xi_h, hu_h, R)

    return kern(xu, xi, gidx3, sidx3)


def kernel(ufeat, ifeat, cj_user, cj_movie, ci_user, ci_movie, W_r, W_rev,
           ufc_W, ufc_b, ifc_W, ifc_b,
           edge_index_0, edge_index_1, edge_index_2, edge_index_3, edge_index_4):
    edges = [edge_index_0, edge_index_1, edge_index_2, edge_index_3, edge_index_4]
    src = jnp.stack([e[0] for e in edges])
    dst = jnp.stack([e[1] for e in edges])

    def perm(n):
        return PK * (n % NQ) + n // NQ

    psrc = perm(src)
    pdst = perm(dst)
    offs = (jnp.arange(R, dtype=jnp.int32) * TNU)[:, None]
    gidx = jnp.concatenate([psrc + offs, pdst + offs], axis=0)
    sidx = jnp.concatenate([pdst, psrc], axis=0)
    padg = jnp.zeros((2 * R, PAD), jnp.int32)
    pads = jnp.broadcast_to(
        TNU + (jnp.arange(PAD, dtype=jnp.int32) % CH), (2 * R, PAD))
    gidx3 = jnp.concatenate([gidx, padg], axis=1).reshape(2 * R * NS, NCHT, CH)
    sidx3 = jnp.concatenate([sidx, pads], axis=1).reshape(2 * R * NS, NCHT, CH)

    eye = jnp.eye(PK, dtype=jnp.float32)[None, :, None, :, None]
    wblk_u = (eye * W_r[:, None, :, None, :]).reshape(R, PK * D_IN, PK * MSG_R)
    wblk_i = (eye * W_rev[:, None, :, None, :]).reshape(R, PK * D_IN, PK * MSG_R)
    eyeg = jnp.eye(PK, dtype=jnp.float32)[None, :, :, None, None]
    wfc_u = (eyeg * ufc_W.reshape(R, MSG_R, OUT)[:, None, None, :, :]
             ).reshape(R, PK, PK * MSG_R, OUT)
    wfc_i = (eyeg * ifc_W.reshape(R, MSG_R, OUT)[:, None, None, :, :]
             ).reshape(R, PK, PK * MSG_R, OUT)
    zpad = jnp.zeros((TNU - NU, 1), jnp.float32)
    cjp_u = jnp.repeat(
        jnp.concatenate([cj_user, zpad]).reshape(PK, NQ).T, MSG_R, axis=1)
    cjp_i = jnp.repeat(
        jnp.concatenate([cj_movie, zpad]).reshape(PK, NQ).T, MSG_R, axis=1)

    xu = _project(ufeat, wblk_u, cjp_u)
    xi = _project(ifeat, wblk_i, cjp_i)

    hi, hu = _sc_segment_sums(
        xu.reshape(R * TNU, MSG_R), xi.reshape(R * TNU, MSG_R), gidx3, sidx3)

    u_out = _fc(hu.reshape(R * NQ, PK * MSG_R), wfc_u, ci_user,
                ufc_b.reshape(1, OUT))
    i_out = _fc(hi.reshape(R * NQ, PK * MSG_R), wfc_i, ci_movie,
                ifc_b.reshape(1, OUT))
    return (u_out, i_out)
